# Initial kernel scaffold; baseline (speedup 1.0000x reference)
import jax
import jax.numpy as jnp
from jax.experimental import pallas as pl


def kernel(x, w_gate, w_noise, w_expert):
    raise NotImplementedError("write your pallas kernel here")


# speedup vs baseline: 91.7981x; 91.7981x over previous
import functools

import jax
import jax.numpy as jnp
import numpy as np
from jax.experimental import pallas as pl
from jax.experimental.pallas import tpu as pltpu

E = 8
TOP_K = 2
D = 1024
N = 4096
BT = 256
NT = N // BT

_EPS = float(np.finfo(float).eps)


def _moe_dense_body(x_ref, wg_ref, we_ref, out_ref, cv_ref, imp_ref):
    t = pl.program_id(0)
    x = x_ref[...]
    logits = jnp.dot(x, wg_ref[...], preferred_element_type=jnp.float32)

    cols = jax.lax.broadcasted_iota(jnp.int32, (BT, E), 1)
    i1 = jnp.argmax(logits, axis=1)
    v1 = jnp.max(logits, axis=1)
    masked = jnp.where(cols == i1[:, None], -jnp.inf, logits)
    i2 = jnp.argmax(masked, axis=1)
    v2 = jnp.max(masked, axis=1)
    ex = jnp.exp(v2 - v1)
    denom = 1.0 + ex
    g1 = 1.0 / denom
    g2 = ex / denom
    gates = (jnp.where(cols == i1[:, None], g1[:, None], 0.0)
             + jnp.where(cols == i2[:, None], g2[:, None], 0.0))

    @pl.when(t == 0)
    def _():
        imp_ref[...] = jnp.zeros_like(imp_ref)

    imp_ref[...] += jnp.sum(gates, axis=0)[None, :]

    acc = jnp.zeros((BT, D), jnp.float32)
    for e in range(E):
        h = jnp.dot(x, we_ref[e], preferred_element_type=jnp.float32)
        h = jnp.maximum(h, 0.0)
        g_e = gates[:, e][:, None]
        acc = acc + jnp.where(g_e > 0.0, g_e * jnp.exp(h), 0.0)
    acc = jnp.where(acc == 0.0, _EPS, acc)
    out_ref[...] = jnp.log(acc)

    @pl.when(t == NT - 1)
    def _():
        imp = imp_ref[0, :]
        m = jnp.mean(imp)
        var = jnp.mean((imp - m) ** 2)
        cv_ref[...] = (var / (m * m + 1e-10)).reshape(1, 1)


@functools.partial(jax.jit)
def _moe_dense(x, w_gate, w_expert):
    out, cv = pl.pallas_call(
        _moe_dense_body,
        grid=(NT,),
        in_specs=[
            pl.BlockSpec((BT, D), lambda t: (t, 0)),
            pl.BlockSpec((D, E), lambda t: (0, 0)),
            pl.BlockSpec((E, D, D), lambda t: (0, 0, 0)),
        ],
        out_specs=[
            pl.BlockSpec((BT, D), lambda t: (t, 0)),
            pl.BlockSpec((1, 1), lambda t: (0, 0)),
        ],
        out_shape=[
            jax.ShapeDtypeStruct((N, D), jnp.float32),
            jax.ShapeDtypeStruct((1, 1), jnp.float32),
        ],
        scratch_shapes=[pltpu.VMEM((1, E), jnp.float32)],
    )(x, w_gate, w_expert)
    return out, cv[0, 0]


def kernel(x, w_gate, w_noise, w_expert):
    del w_noise
    return _moe_dense(x, w_gate, w_expert)
